# baseline (speedup 1.0000x reference)
import functools

import jax
import jax.numpy as jnp
from jax import lax
from jax.experimental import pallas as pl
from jax.experimental.pallas import tpu as pltpu
from jax.experimental.pallas import tpu_sc as plsc

D_H = 4096
NUM_ASPECTS = 6
BATCH = 4096

_NC = 2
_NS = 16
_NW = _NC * _NS
_BPW = BATCH // _NW
_C = 8
_NCHUNK = _BPW // _C


@functools.partial(
    pl.kernel,
    mesh=plsc.VectorSubcoreMesh(core_axis_name="c", subcore_axis_name="s"),
    out_type=jax.ShapeDtypeStruct((BATCH, D_H), jnp.float32),
    scratch_types=[
        pltpu.VMEM((_BPW,), jnp.int32),
        pltpu.VMEM((_C, D_H), jnp.float32),
        pltpu.VMEM((_C, D_H), jnp.float32),
        pltpu.SemaphoreType.DMA,
    ],
)
def _lookup(idx_hbm, table_hbm, out_hbm, idx_v, buf0, buf1, sem):
    wid = lax.axis_index("s") * _NC + lax.axis_index("c")
    base = wid * _BPW
    pltpu.sync_copy(idx_hbm.at[pl.ds(base, _BPW)], idx_v)

    bufs = (buf0, buf1)
    pltpu.async_copy(table_hbm.at[idx_v.at[pl.ds(0, _C)]], bufs[0], sem)
    for g in range(_NCHUNK):
        cur = bufs[g % 2]
        pltpu.make_async_copy(table_hbm.at[idx_v.at[pl.ds(g * _C, _C)]],
                              cur, sem).wait()
        if g + 1 < _NCHUNK:
            pltpu.async_copy(
                table_hbm.at[idx_v.at[pl.ds((g + 1) * _C, _C)]],
                bufs[(g + 1) % 2], sem)
        pltpu.sync_copy(cur, out_hbm.at[pl.ds(base + g * _C, _C)])


def kernel(aspect_idx, embed_weight):
    return _lookup(aspect_idx.astype(jnp.int32), embed_weight)


# speedup vs baseline: 2.8993x; 2.8993x over previous
import functools

import jax
import jax.numpy as jnp
from jax import lax
from jax.experimental import pallas as pl
from jax.experimental.pallas import tpu as pltpu
from jax.experimental.pallas import tpu_sc as plsc

D_H = 4096
NUM_ASPECTS = 6
BATCH = 4096

_NC = 2
_NS = 16
_NW = _NC * _NS
_BPW = BATCH // _NW
_L = 16
_NGRP = _BPW // _L


@functools.partial(
    pl.kernel,
    mesh=plsc.VectorSubcoreMesh(core_axis_name="c", subcore_axis_name="s"),
    out_type=jax.ShapeDtypeStruct((BATCH, D_H), jnp.float32),
    scratch_types=[
        pltpu.VMEM((_BPW,), jnp.int32),
        pltpu.VMEM((NUM_ASPECTS, D_H), jnp.float32),
        pltpu.SemaphoreType.DMA,
    ],
)
def _lookup(idx_hbm, table_hbm, out_hbm, idx_v, table_v, sem):
    wid = lax.axis_index("s") * _NC + lax.axis_index("c")
    base = wid * _BPW
    pltpu.sync_copy(table_hbm, table_v)
    pltpu.sync_copy(idx_hbm.at[pl.ds(base, _BPW)], idx_v)

    copies = []
    for g in range(_NGRP):
        idx16 = idx_v[pl.ds(g * _L, _L)]
        for j in range(_L):
            sj = idx16[j]
            row = base + g * _L + j
            copies.append(pltpu.make_async_copy(
                table_v.at[pl.ds(sj, 1)], out_hbm.at[pl.ds(row, 1)], sem))
    for c in copies:
        c.start()
    for c in copies:
        c.wait()


def kernel(aspect_idx, embed_weight):
    return _lookup(aspect_idx.astype(jnp.int32), embed_weight)
